# baseline (speedup 1.0000x reference)
import functools

import jax
import jax.numpy as jnp
from jax import lax
from jax.experimental import pallas as pl
from jax.experimental.pallas import tpu as pltpu
from jax.experimental.pallas import tpu_sc as plsc

X_DIM = 512
Y_DIM = 512
EMBED_DIM = 128

_NC = 2
_NS = 16
_NW = _NC * _NS
_D_PER_W = EMBED_DIM // _NW
_NSLOT = 8
_YBLK = 16
_NYB = Y_DIM // _YBLK
_NXC = X_DIM // 16
_RUNROLL = 8

_mesh = plsc.VectorSubcoreMesh(core_axis_name="c", subcore_axis_name="s")


@functools.partial(
    pl.kernel,
    mesh=_mesh,
    out_type=jax.ShapeDtypeStruct((EMBED_DIM * Y_DIM, X_DIM), jnp.float32),
    scratch_types=[
        pltpu.VMEM((_D_PER_W, X_DIM), jnp.float32),
        pltpu.VMEM((_D_PER_W, Y_DIM), jnp.float32),
        pltpu.VMEM((_NSLOT, _YBLK, X_DIM), jnp.float32),
    ]
    + [pltpu.SemaphoreType.DMA] * _NSLOT,
    compiler_params=pltpu.CompilerParams(needs_layout_passes=False),
)
def _sc_embed(xt_hbm, yt_hbm, out_hbm, xv, yv, buf, *sems):
    wid = lax.axis_index("s") * _NC + lax.axis_index("c")
    d0 = wid * _D_PER_W

    cpx = pltpu.async_copy(xt_hbm.at[pl.ds(d0, _D_PER_W)], xv, sems[0])
    cpy = pltpu.async_copy(yt_hbm.at[pl.ds(d0, _D_PER_W)], yv, sems[1])
    cpx.wait()
    cpy.wait()

    def drain(slot):
        pltpu.make_async_copy(
            buf.at[slot], out_hbm.at[pl.ds(0, _YBLK)], sems[slot]
        ).wait()

    def do_plane(i, carry):
        xvs = [xv[i, pl.ds(j * 16, 16)] for j in range(_NXC)]
        row_i = jnp.full((16,), i, dtype=jnp.int32)

        def do_group(bb, carry2):
            for slot in range(_NSLOT):
                b = bb * _NSLOT + slot
                first_use = jnp.logical_and(i == 0, bb == 0)

                @pl.when(jnp.logical_not(first_use))
                def _():
                    drain(slot)

                y0 = b * _YBLK

                def fill_rows(t, c):
                    ys = []
                    for r in range(_RUNROLL):
                        y = t * _RUNROLL + r
                        idx = jnp.full((16,), y0 + y, dtype=jnp.int32)
                        ys.append(plsc.load_gather(yv, [row_i, idx]))
                    for r in range(_RUNROLL):
                        y = t * _RUNROLL + r
                        for j in range(_NXC):
                            buf[slot, y, pl.ds(j * 16, 16)] = xvs[j] + ys[r]
                    return c

                lax.fori_loop(0, _YBLK // _RUNROLL, fill_rows, 0)

                row0 = (d0 + i) * Y_DIM + y0
                pltpu.async_copy(
                    buf.at[slot], out_hbm.at[pl.ds(row0, _YBLK)], sems[slot]
                )
            return carry2

        lax.fori_loop(0, _NYB // _NSLOT, do_group, 0)
        return carry

    lax.fori_loop(0, _D_PER_W, do_plane, 0)
    for slot in range(_NSLOT):
        drain(slot)


def kernel(x_table, y_table):
    xT = x_table.T
    yT = y_table.T
    out = _sc_embed(xT, yT)
    return out.reshape(1, EMBED_DIM, Y_DIM, X_DIM)


# speedup vs baseline: 1.3684x; 1.3684x over previous
import functools

import jax
import jax.numpy as jnp
from jax import lax
from jax.experimental import pallas as pl
from jax.experimental.pallas import tpu as pltpu
from jax.experimental.pallas import tpu_sc as plsc

X_DIM = 512
Y_DIM = 512
EMBED_DIM = 128

_NC = 2
_NS = 16
_NW = _NC * _NS
_D_PER_W = EMBED_DIM // _NW
_NSLOT = 2
_YBLK = 64
_NYB = Y_DIM // _YBLK
_NXC = X_DIM // 16
_RUNROLL = 8

_mesh = plsc.VectorSubcoreMesh(core_axis_name="c", subcore_axis_name="s")


@functools.partial(
    pl.kernel,
    mesh=_mesh,
    out_type=jax.ShapeDtypeStruct((EMBED_DIM * Y_DIM, X_DIM), jnp.float32),
    scratch_types=[
        pltpu.VMEM((_D_PER_W, X_DIM), jnp.float32),
        pltpu.VMEM((_D_PER_W, Y_DIM), jnp.float32),
        pltpu.VMEM((_NSLOT, _YBLK, X_DIM), jnp.float32),
    ]
    + [pltpu.SemaphoreType.DMA] * _NSLOT,
    compiler_params=pltpu.CompilerParams(needs_layout_passes=False),
)
def _sc_embed(xt_hbm, yt_hbm, out_hbm, xv, yv, buf, *sems):
    wid = lax.axis_index("s") * _NC + lax.axis_index("c")
    d0 = wid * _D_PER_W

    cpx = pltpu.async_copy(xt_hbm.at[pl.ds(d0, _D_PER_W)], xv, sems[0])
    cpy = pltpu.async_copy(yt_hbm.at[pl.ds(d0, _D_PER_W)], yv, sems[1])
    cpx.wait()
    cpy.wait()

    def drain(slot):
        pltpu.make_async_copy(
            buf.at[slot], out_hbm.at[pl.ds(0, _YBLK)], sems[slot]
        ).wait()

    def do_plane(i, carry):
        xvs = [xv[i, pl.ds(j * 16, 16)] for j in range(_NXC)]
        row_i = jnp.full((16,), i, dtype=jnp.int32)

        def do_group(bb, carry2):
            for slot in range(_NSLOT):
                b = bb * _NSLOT + slot
                first_use = jnp.logical_and(i == 0, bb == 0)

                @pl.when(jnp.logical_not(first_use))
                def _():
                    drain(slot)

                y0 = b * _YBLK

                def fill_rows(t, c):
                    ys = []
                    for r in range(_RUNROLL):
                        y = t * _RUNROLL + r
                        idx = jnp.full((16,), y0 + y, dtype=jnp.int32)
                        ys.append(plsc.load_gather(yv, [row_i, idx]))
                    for r in range(_RUNROLL):
                        y = t * _RUNROLL + r
                        for j in range(_NXC):
                            buf[slot, y, pl.ds(j * 16, 16)] = xvs[j] + ys[r]
                    return c

                lax.fori_loop(0, _YBLK // _RUNROLL, fill_rows, 0)

                row0 = (d0 + i) * Y_DIM + y0
                pltpu.async_copy(
                    buf.at[slot], out_hbm.at[pl.ds(row0, _YBLK)], sems[slot]
                )
            return carry2

        lax.fori_loop(0, _NYB // _NSLOT, do_group, 0)
        return carry

    lax.fori_loop(0, _D_PER_W, do_plane, 0)
    for slot in range(_NSLOT):
        drain(slot)


def kernel(x_table, y_table):
    xT = x_table.T
    yT = y_table.T
    out = _sc_embed(xT, yT)
    return out.reshape(1, EMBED_DIM, Y_DIM, X_DIM)


# speedup vs baseline: 1.3748x; 1.0047x over previous
import functools

import jax
import jax.numpy as jnp
from jax import lax
from jax.experimental import pallas as pl
from jax.experimental.pallas import tpu as pltpu
from jax.experimental.pallas import tpu_sc as plsc

X_DIM = 512
Y_DIM = 512
EMBED_DIM = 128

_NC = 2
_NS = 16
_NW = _NC * _NS
_D_PER_W = EMBED_DIM // _NW
_NSLOT = 2
_YBLK = 64
_NYB = Y_DIM // _YBLK
_NXC = X_DIM // 16
_RUNROLL = 8

_mesh = plsc.VectorSubcoreMesh(core_axis_name="c", subcore_axis_name="s")


@functools.partial(
    pl.kernel,
    mesh=_mesh,
    out_type=jax.ShapeDtypeStruct((EMBED_DIM * Y_DIM, X_DIM), jnp.float32),
    scratch_types=[
        pltpu.VMEM((_D_PER_W, X_DIM), jnp.float32),
        pltpu.VMEM((_D_PER_W, Y_DIM), jnp.float32),
        pltpu.VMEM((_NSLOT, _YBLK, X_DIM), jnp.float32),
    ]
    + [pltpu.SemaphoreType.DMA] * _NSLOT,
    compiler_params=pltpu.CompilerParams(
        needs_layout_passes=False, skip_device_barrier=True
    ),
)
def _sc_embed(xt_hbm, yt_hbm, out_hbm, xv, yv, buf, *sems):
    wid = lax.axis_index("s") * _NC + lax.axis_index("c")
    d0 = wid * _D_PER_W

    cpx = pltpu.async_copy(xt_hbm.at[pl.ds(d0, _D_PER_W)], xv, sems[0])
    cpy = pltpu.async_copy(yt_hbm.at[pl.ds(d0, _D_PER_W)], yv, sems[1])
    cpx.wait()
    cpy.wait()

    def drain(slot):
        pltpu.make_async_copy(
            buf.at[slot], out_hbm.at[pl.ds(0, _YBLK)], sems[slot]
        ).wait()

    def do_plane(i, carry):
        xvs = [xv[i, pl.ds(j * 16, 16)] for j in range(_NXC)]
        row_i = jnp.full((16,), i, dtype=jnp.int32)

        def do_group(bb, carry2):
            for slot in range(_NSLOT):
                b = bb * _NSLOT + slot
                first_use = jnp.logical_and(i == 0, bb == 0)

                @pl.when(jnp.logical_not(first_use))
                def _():
                    drain(slot)

                y0 = b * _YBLK

                def fill_rows(t, c):
                    ys = []
                    for r in range(_RUNROLL):
                        y = t * _RUNROLL + r
                        idx = jnp.full((16,), y0 + y, dtype=jnp.int32)
                        ys.append(plsc.load_gather(yv, [row_i, idx]))
                    for r in range(_RUNROLL):
                        y = t * _RUNROLL + r
                        for j in range(_NXC):
                            buf[slot, y, pl.ds(j * 16, 16)] = xvs[j] + ys[r]
                    return c

                lax.fori_loop(0, _YBLK // _RUNROLL, fill_rows, 0)

                row0 = (d0 + i) * Y_DIM + y0
                pltpu.async_copy(
                    buf.at[slot], out_hbm.at[pl.ds(row0, _YBLK)], sems[slot]
                )
            return carry2

        lax.fori_loop(0, _NYB // _NSLOT, do_group, 0)
        return carry

    lax.fori_loop(0, _D_PER_W, do_plane, 0)
    for slot in range(_NSLOT):
        drain(slot)


def kernel(x_table, y_table):
    xT = x_table.T
    yT = y_table.T
    out = _sc_embed(xT, yT)
    return out.reshape(1, EMBED_DIM, Y_DIM, X_DIM)
